# Initial kernel scaffold; baseline (speedup 1.0000x reference)
import jax
import jax.numpy as jnp
from jax.experimental import pallas as pl


def kernel(u, v, user_tables, item_tables, r_weight):
    raise NotImplementedError("write your pallas kernel here")


# speedup vs baseline: 1.0134x; 1.0134x over previous
import functools

import jax
import jax.numpy as jnp
from jax import lax
from jax.experimental import pallas as pl
from jax.experimental.pallas import tpu as pltpu
from jax.experimental.pallas import tpu_sc as plsc

N_REL = 8
D = 32
NC = 2
NS = 16
L = 16
NW = NC * NS


def _rsqrt(x):
    i = plsc.bitcast(x, jnp.int32)
    y = plsc.bitcast(jnp.int32(0x5F3759DF) - (i >> 1), jnp.float32)
    for _ in range(3):
        y = y * (1.5 - 0.5 * x * y * y)
    return y


def kernel(u, v, user_tables, item_tables, r_weight):
    B = u.shape[0]
    n_users = user_tables.shape[1]
    n_items = item_tables.shape[1]
    b_per_w = B // NW
    n_chunks = b_per_w // L

    ut = user_tables.reshape(N_REL * n_users, D)
    it = item_tables.reshape(N_REL * n_items, D)
    w = jnp.pad(r_weight.reshape(-1), (0, L - N_REL))

    mesh = plsc.VectorSubcoreMesh(core_axis_name="c", subcore_axis_name="s")

    @functools.partial(
        pl.kernel,
        mesh=mesh,
        out_type=jax.ShapeDtypeStruct((B,), jnp.float32),
        compiler_params=pltpu.CompilerParams(
            needs_layout_passes=False, use_tc_tiling_on_sc=False),
        scratch_types=[
            pltpu.VMEM((b_per_w,), jnp.int32),
            pltpu.VMEM((b_per_w,), jnp.int32),
            pltpu.VMEM((b_per_w,), jnp.int32),
            pltpu.VMEM((b_per_w,), jnp.int32),
            pltpu.VMEM((b_per_w, D), jnp.float32),
            pltpu.VMEM((b_per_w, D), jnp.float32),
            pltpu.VMEM((b_per_w,), jnp.float32),
            pltpu.VMEM((L,), jnp.float32),
            pltpu.SemaphoreType.DMA,
            pltpu.SemaphoreType.DMA,
        ],
    )
    def _per_kernel(u_hbm, v_hbm, ut_hbm, it_hbm, w_hbm, out_hbm,
                    iu, iv, ou, ov, ru, rv, acc, wv, sem_u, sem_v):
        wid = lax.axis_index("s") * NC + lax.axis_index("c")
        base = wid * b_per_w
        pltpu.sync_copy(u_hbm.at[pl.ds(base, b_per_w)], iu)
        pltpu.sync_copy(v_hbm.at[pl.ds(base, b_per_w)], iv)
        pltpu.sync_copy(w_hbm, wv)

        wall = wv[pl.ds(0, L)]

        def zero_body(c, carry):
            acc[pl.ds(c * L, L)] = jnp.zeros((L,), jnp.float32)
            return carry
        lax.fori_loop(0, n_chunks, zero_body, 0)

        for r in range(N_REL):
            def off_body(c, carry, _r=r):
                s = pl.ds(c * L, L)
                ou[s] = iu[s] + _r * n_users
                ov[s] = iv[s] + _r * n_items
                return carry
            lax.fori_loop(0, n_chunks, off_body, 0)

            cu = pltpu.async_copy(ut_hbm.at[ou], ru, sem_u)
            cv = pltpu.async_copy(it_hbm.at[ov], rv, sem_v)
            cu.wait()
            cv.wait()

            wr = wall[r]

            def comp_body(c, carry):
                row_ids = c * L + lax.iota(jnp.int32, L)
                duv = jnp.zeros((L,), jnp.float32)
                su = jnp.zeros((L,), jnp.float32)
                sv = jnp.zeros((L,), jnp.float32)
                for d in range(D):
                    col = jnp.full((L,), d, jnp.int32)
                    eu = plsc.load_gather(ru, [row_ids, col])
                    ev = plsc.load_gather(rv, [row_ids, col])
                    duv = duv + eu * ev
                    su = su + eu * eu
                    sv = sv + ev * ev
                denom = jnp.maximum(su, 1.0) * jnp.maximum(sv, 1.0)
                val = duv * _rsqrt(denom) * wr
                s = pl.ds(c * L, L)
                acc[s] = acc[s] + val
                return carry
            lax.fori_loop(0, n_chunks, comp_body, 0)

        def fin_body(c, carry):
            s = pl.ds(c * L, L)
            x = acc[s]
            acc[s] = 1.0 / (1.0 + jnp.exp(-x))
            return carry
        lax.fori_loop(0, n_chunks, fin_body, 0)
        pltpu.sync_copy(acc, out_hbm.at[pl.ds(base, b_per_w)])

    return _per_kernel(u, v, ut, it, w)


# speedup vs baseline: 1.0152x; 1.0018x over previous
import functools

import jax
import jax.numpy as jnp
from jax import lax
from jax.experimental import pallas as pl
from jax.experimental.pallas import tpu as pltpu
from jax.experimental.pallas import tpu_sc as plsc

N_REL = 8
D = 32
NC = 2
NS = 16
L = 16
NW = NC * NS


def _rsqrt(x):
    i = plsc.bitcast(x, jnp.int32)
    y = plsc.bitcast(jnp.int32(0x5F3759DF) - (i >> 1), jnp.float32)
    for _ in range(3):
        y = y * (1.5 - 0.5 * x * y * y)
    return y


def kernel(u, v, user_tables, item_tables, r_weight):
    B = u.shape[0]
    b_per_w = B // NW
    n_chunks = b_per_w // L

    w = jnp.pad(r_weight.reshape(-1), (0, L - N_REL))

    mesh = plsc.VectorSubcoreMesh(core_axis_name="c", subcore_axis_name="s")

    @functools.partial(
        pl.kernel,
        mesh=mesh,
        out_type=jax.ShapeDtypeStruct((B,), jnp.float32),
        compiler_params=pltpu.CompilerParams(
            needs_layout_passes=False, use_tc_tiling_on_sc=False),
        scratch_types=[
            pltpu.VMEM((b_per_w,), jnp.int32),
            pltpu.VMEM((b_per_w,), jnp.int32),
            pltpu.VMEM((b_per_w, D), jnp.float32),
            pltpu.VMEM((b_per_w, D), jnp.float32),
            pltpu.VMEM((b_per_w,), jnp.float32),
            pltpu.VMEM((L,), jnp.float32),
            pltpu.SemaphoreType.DMA,
            pltpu.SemaphoreType.DMA,
        ],
    )
    def _per_kernel(u_hbm, v_hbm, ut_hbm, it_hbm, w_hbm, out_hbm,
                    iu, iv, ru, rv, acc, wv, sem_u, sem_v):
        wid = lax.axis_index("s") * NC + lax.axis_index("c")
        base = wid * b_per_w
        pltpu.sync_copy(u_hbm.at[pl.ds(base, b_per_w)], iu)
        pltpu.sync_copy(v_hbm.at[pl.ds(base, b_per_w)], iv)
        pltpu.sync_copy(w_hbm, wv)

        wall = wv[pl.ds(0, L)]

        def zero_body(c, carry):
            acc[pl.ds(c * L, L)] = jnp.zeros((L,), jnp.float32)
            return carry
        lax.fori_loop(0, n_chunks, zero_body, 0)

        for r in range(N_REL):
            cu = pltpu.async_copy(ut_hbm.at[r].at[iu], ru, sem_u)
            cv = pltpu.async_copy(it_hbm.at[r].at[iv], rv, sem_v)
            cu.wait()
            cv.wait()

            wr = wall[r]

            def comp_body(c, carry):
                row_ids = c * L + lax.iota(jnp.int32, L)
                duv = jnp.zeros((L,), jnp.float32)
                su = jnp.zeros((L,), jnp.float32)
                sv = jnp.zeros((L,), jnp.float32)
                for d in range(D):
                    col = jnp.full((L,), d, jnp.int32)
                    eu = plsc.load_gather(ru, [row_ids, col])
                    ev = plsc.load_gather(rv, [row_ids, col])
                    duv = duv + eu * ev
                    su = su + eu * eu
                    sv = sv + ev * ev
                denom = jnp.maximum(su, 1.0) * jnp.maximum(sv, 1.0)
                val = duv * _rsqrt(denom) * wr
                s = pl.ds(c * L, L)
                acc[s] = acc[s] + val
                return carry
            lax.fori_loop(0, n_chunks, comp_body, 0)

        def fin_body(c, carry):
            s = pl.ds(c * L, L)
            x = acc[s]
            acc[s] = 1.0 / (1.0 + jnp.exp(-x))
            return carry
        lax.fori_loop(0, n_chunks, fin_body, 0)
        pltpu.sync_copy(acc, out_hbm.at[pl.ds(base, b_per_w)])

    return _per_kernel(u, v, user_tables, item_tables, w)


# speedup vs baseline: 1.3932x; 1.3724x over previous
import functools

import jax
import jax.numpy as jnp
from jax import lax
from jax.experimental import pallas as pl
from jax.experimental.pallas import tpu as pltpu
from jax.experimental.pallas import tpu_sc as plsc

N_REL = 8
D = 32
NC = 2
NS = 16
L = 16
NW = NC * NS


def _rsqrt(x):
    i = plsc.bitcast(x, jnp.int32)
    y = plsc.bitcast(jnp.int32(0x5F3759DF) - (i >> 1), jnp.float32)
    for _ in range(3):
        y = y * (1.5 - 0.5 * x * y * y)
    return y


def kernel(u, v, user_tables, item_tables, r_weight):
    B = u.shape[0]
    b_per_w = B // NW
    n_chunks = b_per_w // L

    ut = user_tables.transpose(0, 2, 1)
    it = item_tables.transpose(0, 2, 1)
    w = jnp.pad(r_weight.reshape(-1), (0, L - N_REL))

    mesh = plsc.VectorSubcoreMesh(core_axis_name="c", subcore_axis_name="s")

    @functools.partial(
        pl.kernel,
        mesh=mesh,
        out_type=jax.ShapeDtypeStruct((B,), jnp.float32),
        compiler_params=pltpu.CompilerParams(
            needs_layout_passes=False, use_tc_tiling_on_sc=False),
        scratch_types=[
            pltpu.VMEM((b_per_w,), jnp.int32),
            pltpu.VMEM((b_per_w,), jnp.int32),
            pltpu.VMEM((2, D, b_per_w), jnp.float32),
            pltpu.VMEM((2, D, b_per_w), jnp.float32),
            pltpu.VMEM((b_per_w,), jnp.float32),
            pltpu.VMEM((L,), jnp.float32),
            pltpu.SemaphoreType.DMA,
            pltpu.SemaphoreType.DMA,
            pltpu.SemaphoreType.DMA,
            pltpu.SemaphoreType.DMA,
        ],
    )
    def _per_kernel(u_hbm, v_hbm, ut_hbm, it_hbm, w_hbm, out_hbm,
                    iu, iv, eu, ev, acc, wv, s_u0, s_v0, s_u1, s_v1):
        wid = lax.axis_index("s") * NC + lax.axis_index("c")
        base = wid * b_per_w
        pltpu.sync_copy(u_hbm.at[pl.ds(base, b_per_w)], iu)
        pltpu.sync_copy(v_hbm.at[pl.ds(base, b_per_w)], iv)
        pltpu.sync_copy(w_hbm, wv)

        sems_u = (s_u0, s_u1)
        sems_v = (s_v0, s_v1)

        def fire(r, slot):
            su, sv = sems_u[slot], sems_v[slot]

            def fire_body(d, carry):
                pltpu.async_copy(ut_hbm.at[r, d].at[iu], eu.at[slot, d], su)
                pltpu.async_copy(it_hbm.at[r, d].at[iv], ev.at[slot, d], sv)
                return carry
            lax.fori_loop(0, D, fire_body, 0)

        def drain(r, slot):
            su, sv = sems_u[slot], sems_v[slot]

            def drain_body(d, carry):
                pltpu.make_async_copy(
                    ut_hbm.at[r, d].at[iu], eu.at[slot, d], su).wait()
                pltpu.make_async_copy(
                    it_hbm.at[r, d].at[iv], ev.at[slot, d], sv).wait()
                return carry
            lax.fori_loop(0, D, drain_body, 0)

        wall = wv[pl.ds(0, L)]

        fire(0, 0)
        for r in range(N_REL):
            slot = r % 2
            if r + 1 < N_REL:
                fire(r + 1, 1 - slot)
            drain(r, slot)
            wr = wall[r]

            def comp_body(c, carry, _slot=slot):
                s = pl.ds(c * L, L)
                duv = jnp.zeros((L,), jnp.float32)
                su_a = jnp.zeros((L,), jnp.float32)
                sv_a = jnp.zeros((L,), jnp.float32)
                for d in range(D):
                    xu = eu[_slot, d, s]
                    xv = ev[_slot, d, s]
                    duv = duv + xu * xv
                    su_a = su_a + xu * xu
                    sv_a = sv_a + xv * xv
                denom = jnp.maximum(su_a, 1.0) * jnp.maximum(sv_a, 1.0)
                val = duv * _rsqrt(denom) * wr
                if r == 0:
                    acc[s] = val
                else:
                    acc[s] = acc[s] + val
                return carry
            lax.fori_loop(0, n_chunks, comp_body, 0)

        def fin_body(c, carry):
            s = pl.ds(c * L, L)
            x = acc[s]
            acc[s] = 1.0 / (1.0 + jnp.exp(-x))
            return carry
        lax.fori_loop(0, n_chunks, fin_body, 0)
        pltpu.sync_copy(acc, out_hbm.at[pl.ds(base, b_per_w)])

    return _per_kernel(u, v, ut, it, w)
